# baseline (speedup 1.0000x reference)
import jax
import jax.numpy as jnp
from jax.experimental import pallas as pl
from jax.experimental.pallas import tpu as pltpu

N0 = 4096
N1 = 4096
C_IN = 256
C_OUT = 256
CH = 256
NCH = N1 // CH
NBUF = 3


def _body(x0_hbm, b_hbm, w_hbm, out_hbm,
          x0v, wv, xw, bbuf, obuf, ld_sem, in_sem, out_sem):
    for s in range(2):
        pltpu.make_async_copy(
            b_hbm.at[pl.ds(s * CH, CH), :], bbuf.at[s], in_sem.at[s]
        ).start()
    pltpu.make_async_copy(x0_hbm, x0v, ld_sem.at[0]).start()
    pltpu.make_async_copy(w_hbm, wv, ld_sem.at[1]).start()
    pltpu.make_async_copy(x0_hbm, x0v, ld_sem.at[0]).wait()
    pltpu.make_async_copy(w_hbm, wv, ld_sem.at[1]).wait()
    xw[...] = jnp.dot(x0v[...], wv[...], preferred_element_type=jnp.float32)

    for c in range(NCH):
        slot = c % NBUF
        if c + 2 < NCH:
            nslot = (c + 2) % NBUF
            pltpu.make_async_copy(
                b_hbm.at[pl.ds((c + 2) * CH, CH), :],
                bbuf.at[nslot],
                in_sem.at[nslot],
            ).start()
        pltpu.make_async_copy(
            b_hbm.at[pl.ds(c * CH, CH), :], bbuf.at[slot], in_sem.at[slot]
        ).wait()
        acc = jnp.dot(bbuf[slot], xw[...], preferred_element_type=jnp.float32)
        oslot = c % 2
        if c >= 2:
            pltpu.make_async_copy(
                obuf.at[oslot],
                out_hbm.at[pl.ds((c - 2) * CH, CH), :],
                out_sem.at[oslot],
            ).wait()
        obuf[oslot] = jnp.where(acc > 0, acc, jnp.exp(jnp.minimum(acc, 0.0)) - 1.0)
        pltpu.make_async_copy(
            obuf.at[oslot], out_hbm.at[pl.ds(c * CH, CH), :], out_sem.at[oslot]
        ).start()

    for c in (NCH - 2, NCH - 1):
        pltpu.make_async_copy(
            obuf.at[c % 2], out_hbm.at[pl.ds(c * CH, CH), :], out_sem.at[c % 2]
        ).wait()


def kernel(x_0, neighborhood_0_to_1, W):
    return pl.pallas_call(
        _body,
        in_specs=[
            pl.BlockSpec(memory_space=pl.ANY),
            pl.BlockSpec(memory_space=pl.ANY),
            pl.BlockSpec(memory_space=pl.ANY),
        ],
        out_specs=pl.BlockSpec(memory_space=pl.ANY),
        out_shape=jax.ShapeDtypeStruct((N1, C_OUT), jnp.float32),
        scratch_shapes=[
            pltpu.VMEM((N0, C_IN), jnp.float32),
            pltpu.VMEM((C_IN, C_OUT), jnp.float32),
            pltpu.VMEM((N0, C_OUT), jnp.float32),
            pltpu.VMEM((NBUF, CH, N0), jnp.float32),
            pltpu.VMEM((2, CH, C_OUT), jnp.float32),
            pltpu.SemaphoreType.DMA((2,)),
            pltpu.SemaphoreType.DMA((NBUF,)),
            pltpu.SemaphoreType.DMA((2,)),
        ],
    )(x_0, neighborhood_0_to_1, W)
